# baseline (speedup 1.0000x reference)
import functools

import jax
import jax.numpy as jnp
from jax import lax
from jax.experimental import pallas as pl
from jax.experimental.pallas import tpu as pltpu
from jax.experimental.pallas import tpu_sc as plsc

N_NODES = 10000
N_EDGES = 320000
D = 128
EPS = 1e-5

NC = 2
NS = 16
NW = NC * NS
E_PER_TILE = N_EDGES // NW
K = 96
NCHUNK = E_PER_TILE // K
KT = E_PER_TILE - NCHUNK * K
NR = 3
ROWS_MAIN = 624
ROWS_TAIL_OFF = ROWS_MAIN * NS
ROWS_TAIL = N_NODES - ROWS_TAIL_OFF


def _sc_aggregate(edges_hbm, x_hbm, zeros_hbm, out_hbm,
                  iv0, iv1, iv2, dv0, dv1, dv2,
                  rows0, rows1, rows2, iv_t, dv_t, aggr_sh,
                  is0, is1, is2, ds0, ds1, ds2, gs0, gs1, gs2):
    ivs = (iv0, iv1, iv2)
    dvs = (dv0, dv1, dv2)
    isems = (is0, is1, is2)
    dsems = (ds0, ds1, ds2)
    rows = (rows0, rows1, rows2)
    gsems = (gs0, gs1, gs2)

    c = lax.axis_index("c")
    s = lax.axis_index("s")
    wid = s * NC + c
    sbase = wid * E_PER_TILE
    dbase = N_EDGES + wid * E_PER_TILE

    def fire_idx(i, p):
        pltpu.async_copy(edges_hbm.at[pl.ds(sbase + K * i, K)],
                         ivs[p], isems[p])
        pltpu.async_copy(edges_hbm.at[pl.ds(dbase + K * i, K)],
                         dvs[p], dsems[p])

    def drain_isem(p):
        pltpu.make_async_copy(edges_hbm.at[pl.ds(0, K)], ivs[p],
                              isems[p]).wait()

    def drain_dsem(p):
        pltpu.make_async_copy(edges_hbm.at[pl.ds(0, K)], dvs[p],
                              dsems[p]).wait()

    def fire_gather(p):
        pltpu.async_copy(x_hbm.at[ivs[p]], rows[p], gsems[p])

    def drain_gsem(p):
        pltpu.make_async_copy(x_hbm.at[pl.ds(0, K)], rows[p],
                              gsems[p]).wait()

    def scatter(p):
        pltpu.sync_copy(rows[p], aggr_sh.at[dvs[p]], add=True)

    def it(i, im):
        if isinstance(i, int):
            if i + 2 < NCHUNK:
                drain_isem((im + 2) % NR)
                fire_gather((im + 2) % NR)
        else:
            drain_isem((im + 2) % NR)
            fire_gather((im + 2) % NR)
        drain_gsem(im)
        drain_dsem(im)
        scatter(im)
        if isinstance(i, int):
            if i + NR < NCHUNK:
                fire_idx(i + NR, im)
        else:
            fire_idx(i + NR, im)

    for b in range(NR):
        fire_idx(b, b)
    drain_isem(0)
    fire_gather(0)
    drain_isem(1)
    fire_gather(1)

    r0 = s * ROWS_MAIN
    pltpu.sync_copy(zeros_hbm.at[pl.ds(r0, ROWS_MAIN)],
                    aggr_sh.at[pl.ds(r0, ROWS_MAIN)])

    @pl.when(s == NS - 1)
    def _():
        pltpu.sync_copy(zeros_hbm.at[pl.ds(ROWS_TAIL_OFF, ROWS_TAIL)],
                        aggr_sh.at[pl.ds(ROWS_TAIL_OFF, ROWS_TAIL)])

    plsc.subcore_barrier()

    def body(j, carry):
        i0 = j * NR
        for b in range(NR):
            it(i0 + b, b)
        return carry

    lax.fori_loop(0, 33, body, 0)

    for i in range(99, NCHUNK):
        it(i, i % NR)

    pltpu.sync_copy(edges_hbm.at[pl.ds(sbase + NCHUNK * K, KT)], iv_t)
    pltpu.sync_copy(edges_hbm.at[pl.ds(dbase + NCHUNK * K, KT)], dv_t)
    pltpu.async_copy(x_hbm.at[iv_t], rows0.at[pl.ds(0, KT)], gs0).wait()
    pltpu.sync_copy(rows0.at[pl.ds(0, KT)], aggr_sh.at[dv_t], add=True)

    plsc.subcore_barrier()

    pltpu.sync_copy(aggr_sh.at[pl.ds(r0, ROWS_MAIN)],
                    out_hbm.at[c, pl.ds(r0, ROWS_MAIN)])

    @pl.when(s == NS - 1)
    def _():
        pltpu.sync_copy(aggr_sh.at[pl.ds(ROWS_TAIL_OFF, ROWS_TAIL)],
                        out_hbm.at[c, pl.ds(ROWS_TAIL_OFF, ROWS_TAIL)])


_sc_aggregate_call = pl.kernel(
    _sc_aggregate,
    out_type=jax.ShapeDtypeStruct((NC, N_NODES, D), jnp.float32),
    mesh=plsc.VectorSubcoreMesh(core_axis_name="c", subcore_axis_name="s",
                                num_cores=NC, num_subcores=NS),
    scratch_types=(
        [pltpu.VMEM((K,), jnp.int32) for _ in range(NR)]
        + [pltpu.VMEM((K,), jnp.int32) for _ in range(NR)]
        + [pltpu.VMEM((K, D), jnp.float32) for _ in range(NR)]
        + [pltpu.VMEM((KT,), jnp.int32), pltpu.VMEM((KT,), jnp.int32)]
        + [pltpu.VMEM_SHARED((N_NODES, D), jnp.float32)]
        + [pltpu.SemaphoreType.DMA for _ in range(3 * NR)]
    ),
)


def _tc_root(x_ref, wrootT_ref, b_ref, out_ref):
    out_ref[...] = (jnp.dot(x_ref[...], wrootT_ref[...],
                            preferred_element_type=jnp.float32)
                    + b_ref[...])


_tc_root_call = pl.pallas_call(
    _tc_root,
    out_shape=jax.ShapeDtypeStruct((N_NODES, D), jnp.float32),
)


def _tc_finish(parts_ref, hroot_ref, wrT_ref, gamma_ref, beta_ref, out_ref):
    aggr = parts_ref[0] + parts_ref[1]
    h = (jnp.dot(aggr, wrT_ref[...], preferred_element_type=jnp.float32)
         + hroot_ref[...])
    mean = jnp.mean(h, axis=0, keepdims=True)
    d = h - mean
    var = jnp.mean(d * d, axis=0, keepdims=True)
    inv = lax.rsqrt(var + EPS)
    out_ref[...] = jnp.maximum(d * inv * gamma_ref[...] + beta_ref[...], 0.0)


_tc_finish_call = pl.pallas_call(
    _tc_finish,
    out_shape=jax.ShapeDtypeStruct((N_NODES, D), jnp.float32),
)


@jax.jit
def kernel(x, edge_index, batch, W_rel, b_rel, W_root, gamma, beta):
    edges = edge_index.astype(jnp.int32).reshape(2 * N_EDGES)
    zeros = jnp.zeros((N_NODES, D), jnp.float32)
    hroot = _tc_root_call(x, W_root.T, b_rel.reshape(1, D))
    parts = _sc_aggregate_call(edges, x, zeros)
    return _tc_finish_call(parts, hroot, W_rel.T,
                           gamma.reshape(1, D), beta.reshape(1, D))


# speedup vs baseline: 1.1655x; 1.1655x over previous
import functools

import jax
import jax.numpy as jnp
from jax import lax
from jax.experimental import pallas as pl
from jax.experimental.pallas import tpu as pltpu
from jax.experimental.pallas import tpu_sc as plsc

N_NODES = 10000
N_EDGES = 320000
D = 128
EPS = 1e-5

NC = 2
NS = 16
NW = NC * NS
E_PER_TILE = N_EDGES // NW
K = 128
NCHUNK = E_PER_TILE // K
KT = E_PER_TILE - NCHUNK * K
NI = 4
NG = 2
ROWS_MAIN = 624
ROWS_TAIL_OFF = ROWS_MAIN * NS
ROWS_TAIL = N_NODES - ROWS_TAIL_OFF


def _sc_aggregate(edges_hbm, x_hbm, out_hbm,
                  iv0, iv1, iv2, iv3, dv0, dv1, dv2, dv3,
                  rows0, rows1, iv_t, dv_t, aggr_sh,
                  is0, is1, is2, is3, ds0, ds1, ds2, ds3, gs0, gs1):
    ivs = (iv0, iv1, iv2, iv3)
    dvs = (dv0, dv1, dv2, dv3)
    isems = (is0, is1, is2, is3)
    dsems = (ds0, ds1, ds2, ds3)
    rows = (rows0, rows1)
    gsems = (gs0, gs1)

    c = lax.axis_index("c")
    s = lax.axis_index("s")
    wid = s * NC + c
    sbase = wid * E_PER_TILE
    dbase = N_EDGES + wid * E_PER_TILE

    def fire_idx(i, p):
        pltpu.async_copy(edges_hbm.at[pl.ds(sbase + K * i, K)],
                         ivs[p], isems[p])
        pltpu.async_copy(edges_hbm.at[pl.ds(dbase + K * i, K)],
                         dvs[p], dsems[p])

    def drain_isem(p):
        pltpu.make_async_copy(edges_hbm.at[pl.ds(0, K)], ivs[p],
                              isems[p]).wait()

    def drain_dsem(p):
        pltpu.make_async_copy(edges_hbm.at[pl.ds(0, K)], dvs[p],
                              dsems[p]).wait()

    def fire_gather(p):
        pltpu.async_copy(x_hbm.at[ivs[p % NI]], rows[p % NG], gsems[p % NG])

    def drain_gsem(p):
        pltpu.make_async_copy(x_hbm.at[pl.ds(0, K)], rows[p % NG],
                              gsems[p % NG]).wait()

    def scatter(p):
        pltpu.sync_copy(rows[p % NG], aggr_sh.at[dvs[p % NI]], add=True)

    def it(i, im, n=NCHUNK):
        if isinstance(i, int):
            if i + 1 < n:
                drain_isem((im + 1) % NI)
                fire_gather(im + 1)
        else:
            drain_isem((im + 1) % NI)
            fire_gather(im + 1)
        drain_gsem(im)
        drain_dsem(im % NI)
        scatter(im)
        if isinstance(i, int):
            if i + NI < n:
                fire_idx(i + NI, im % NI)
        else:
            fire_idx(i + NI, im % NI)

    for b in range(NI):
        fire_idx(b, b)
    drain_isem(0)
    fire_gather(0)

    z16 = jnp.zeros((16,), jnp.float32)

    def zbody(r, carry):
        for j in range(D // 16):
            rows1[r, pl.ds(j * 16, 16)] = z16
        return carry

    lax.fori_loop(0, K, zbody, 0)

    r0 = s * ROWS_MAIN
    for t in range(4):
        pltpu.sync_copy(rows1, aggr_sh.at[pl.ds(r0 + K * t, K)])
    pltpu.sync_copy(rows1.at[pl.ds(0, 112)],
                    aggr_sh.at[pl.ds(r0 + 4 * K, 112)])

    @pl.when(s == NS - 1)
    def _():
        pltpu.sync_copy(rows1.at[pl.ds(0, ROWS_TAIL)],
                        aggr_sh.at[pl.ds(ROWS_TAIL_OFF, ROWS_TAIL)])

    plsc.subcore_barrier()

    def body(j, carry):
        i0 = j * NI
        for b in range(NI):
            it(i0 + b, b)
        return carry

    lax.fori_loop(0, 18, body, 0)

    for i in range(72, NCHUNK):
        it(i, i % NI)

    pltpu.sync_copy(edges_hbm.at[pl.ds(sbase + NCHUNK * K, KT)], iv_t)
    pltpu.sync_copy(edges_hbm.at[pl.ds(dbase + NCHUNK * K, KT)], dv_t)
    pltpu.async_copy(x_hbm.at[iv_t], rows0.at[pl.ds(0, KT)], gs0).wait()
    pltpu.sync_copy(rows0.at[pl.ds(0, KT)], aggr_sh.at[dv_t], add=True)

    plsc.subcore_barrier()

    pltpu.sync_copy(aggr_sh.at[pl.ds(r0, ROWS_MAIN)],
                    out_hbm.at[c, pl.ds(r0, ROWS_MAIN)])

    @pl.when(s == NS - 1)
    def _():
        pltpu.sync_copy(aggr_sh.at[pl.ds(ROWS_TAIL_OFF, ROWS_TAIL)],
                        out_hbm.at[c, pl.ds(ROWS_TAIL_OFF, ROWS_TAIL)])


_sc_aggregate_call = pl.kernel(
    _sc_aggregate,
    out_type=jax.ShapeDtypeStruct((NC, N_NODES, D), jnp.float32),
    mesh=plsc.VectorSubcoreMesh(core_axis_name="c", subcore_axis_name="s",
                                num_cores=NC, num_subcores=NS),
    scratch_types=(
        [pltpu.VMEM((K,), jnp.int32) for _ in range(NI)]
        + [pltpu.VMEM((K,), jnp.int32) for _ in range(NI)]
        + [pltpu.VMEM((K, D), jnp.float32) for _ in range(NG)]
        + [pltpu.VMEM((KT,), jnp.int32), pltpu.VMEM((KT,), jnp.int32)]
        + [pltpu.VMEM_SHARED((N_NODES, D), jnp.float32)]
        + [pltpu.SemaphoreType.DMA for _ in range(2 * NI + NG)]
    ),
)


def _tc_root(x_ref, wrootT_ref, b_ref, out_ref):
    out_ref[...] = (jnp.dot(x_ref[...], wrootT_ref[...],
                            preferred_element_type=jnp.float32)
                    + b_ref[...])


_tc_root_call = pl.pallas_call(
    _tc_root,
    out_shape=jax.ShapeDtypeStruct((N_NODES, D), jnp.float32),
)


def _tc_finish(parts_ref, hroot_ref, wrT_ref, gamma_ref, beta_ref, out_ref):
    aggr = parts_ref[0] + parts_ref[1]
    h = (jnp.dot(aggr, wrT_ref[...], preferred_element_type=jnp.float32)
         + hroot_ref[...])
    mean = jnp.mean(h, axis=0, keepdims=True)
    d = h - mean
    var = jnp.mean(d * d, axis=0, keepdims=True)
    inv = lax.rsqrt(var + EPS)
    out_ref[...] = jnp.maximum(d * inv * gamma_ref[...] + beta_ref[...], 0.0)


_tc_finish_call = pl.pallas_call(
    _tc_finish,
    out_shape=jax.ShapeDtypeStruct((N_NODES, D), jnp.float32),
)


@jax.jit
def kernel(x, edge_index, batch, W_rel, b_rel, W_root, gamma, beta):
    edges = edge_index.astype(jnp.int32).reshape(2 * N_EDGES)
    hroot = _tc_root_call(x, W_root.T, b_rel.reshape(1, D))
    parts = _sc_aggregate_call(edges, x)
    return _tc_finish_call(parts, hroot, W_rel.T,
                           gamma.reshape(1, D), beta.reshape(1, D))


# speedup vs baseline: 1.1772x; 1.0100x over previous
import functools

import jax
import jax.numpy as jnp
from jax import lax
from jax.experimental import pallas as pl
from jax.experimental.pallas import tpu as pltpu
from jax.experimental.pallas import tpu_sc as plsc

N_NODES = 10000
N_EDGES = 320000
D = 128
EPS = 1e-5

NC = 2
NS = 16
NW = NC * NS
E_PER_TILE = N_EDGES // NW
K = 128
NCHUNK = E_PER_TILE // K
KT = E_PER_TILE - NCHUNK * K
NI = 4
NG = 2
ROWS_MAIN = 624
ROWS_TAIL_OFF = ROWS_MAIN * NS
ROWS_TAIL = N_NODES - ROWS_TAIL_OFF


def _sc_aggregate(edges_hbm, x_hbm, out_hbm,
                  iv0, iv1, iv2, iv3, dv0, dv1, dv2, dv3,
                  rows0, rows1, iv_t, dv_t, rows_t, aggr_sh,
                  is0, is1, is2, is3, ds0, ds1, ds2, ds3, gs0, gs1,
                  ist, dst_s, gst):
    ivs = (iv0, iv1, iv2, iv3)
    dvs = (dv0, dv1, dv2, dv3)
    isems = (is0, is1, is2, is3)
    dsems = (ds0, ds1, ds2, ds3)
    rows = (rows0, rows1)
    gsems = (gs0, gs1)

    c = lax.axis_index("c")
    s = lax.axis_index("s")
    wid = s * NC + c
    sbase = wid * E_PER_TILE
    dbase = N_EDGES + wid * E_PER_TILE

    def fire_idx(i, p):
        pltpu.async_copy(edges_hbm.at[pl.ds(sbase + K * i, K)],
                         ivs[p], isems[p])
        pltpu.async_copy(edges_hbm.at[pl.ds(dbase + K * i, K)],
                         dvs[p], dsems[p])

    def drain_isem(p):
        pltpu.make_async_copy(edges_hbm.at[pl.ds(0, K)], ivs[p],
                              isems[p]).wait()

    def drain_dsem(p):
        pltpu.make_async_copy(edges_hbm.at[pl.ds(0, K)], dvs[p],
                              dsems[p]).wait()

    def fire_gather(p):
        pltpu.async_copy(x_hbm.at[ivs[p % NI]], rows[p % NG], gsems[p % NG])

    def drain_gsem(p):
        pltpu.make_async_copy(x_hbm.at[pl.ds(0, K)], rows[p % NG],
                              gsems[p % NG]).wait()

    def scatter(p):
        pltpu.sync_copy(rows[p % NG], aggr_sh.at[dvs[p % NI]], add=True)

    def it(i, im, n=NCHUNK):
        if isinstance(i, int):
            if i + 1 < n:
                drain_isem((im + 1) % NI)
                fire_gather(im + 1)
        else:
            drain_isem((im + 1) % NI)
            fire_gather(im + 1)
        drain_gsem(im)
        drain_dsem(im % NI)
        scatter(im)
        if isinstance(i, int):
            if i + NI < n:
                fire_idx(i + NI, im % NI)
        else:
            fire_idx(i + NI, im % NI)

    for b in range(NI):
        fire_idx(b, b)
    pltpu.async_copy(edges_hbm.at[pl.ds(sbase + NCHUNK * K, KT)], iv_t, ist)
    pltpu.async_copy(edges_hbm.at[pl.ds(dbase + NCHUNK * K, KT)], dv_t, dst_s)
    drain_isem(0)
    fire_gather(0)
    pltpu.make_async_copy(edges_hbm.at[pl.ds(0, KT)], iv_t, ist).wait()
    pltpu.async_copy(x_hbm.at[iv_t], rows_t, gst)

    z16 = jnp.zeros((16,), jnp.float32)

    def zbody(r, carry):
        for j in range(D // 16):
            rows1[r, pl.ds(j * 16, 16)] = z16
        return carry

    lax.fori_loop(0, K, zbody, 0)

    r0 = s * ROWS_MAIN
    for t in range(4):
        pltpu.sync_copy(rows1, aggr_sh.at[pl.ds(r0 + K * t, K)])
    pltpu.sync_copy(rows1.at[pl.ds(0, 112)],
                    aggr_sh.at[pl.ds(r0 + 4 * K, 112)])

    @pl.when(s == NS - 1)
    def _():
        pltpu.sync_copy(rows1.at[pl.ds(0, ROWS_TAIL)],
                        aggr_sh.at[pl.ds(ROWS_TAIL_OFF, ROWS_TAIL)])

    plsc.subcore_barrier()

    def body(j, carry):
        i0 = j * NI
        for b in range(NI):
            it(i0 + b, b)
        return carry

    lax.fori_loop(0, 18, body, 0)

    for i in range(72, NCHUNK):
        it(i, i % NI)

    pltpu.make_async_copy(x_hbm.at[pl.ds(0, KT)], rows_t, gst).wait()
    pltpu.make_async_copy(edges_hbm.at[pl.ds(0, KT)], dv_t, dst_s).wait()
    pltpu.sync_copy(rows_t, aggr_sh.at[dv_t], add=True)

    plsc.subcore_barrier()

    pltpu.sync_copy(aggr_sh.at[pl.ds(r0, ROWS_MAIN)],
                    out_hbm.at[c, pl.ds(r0, ROWS_MAIN)])

    @pl.when(s == NS - 1)
    def _():
        pltpu.sync_copy(aggr_sh.at[pl.ds(ROWS_TAIL_OFF, ROWS_TAIL)],
                        out_hbm.at[c, pl.ds(ROWS_TAIL_OFF, ROWS_TAIL)])


_sc_aggregate_call = pl.kernel(
    _sc_aggregate,
    out_type=jax.ShapeDtypeStruct((NC, N_NODES, D), jnp.float32),
    mesh=plsc.VectorSubcoreMesh(core_axis_name="c", subcore_axis_name="s",
                                num_cores=NC, num_subcores=NS),
    scratch_types=(
        [pltpu.VMEM((K,), jnp.int32) for _ in range(NI)]
        + [pltpu.VMEM((K,), jnp.int32) for _ in range(NI)]
        + [pltpu.VMEM((K, D), jnp.float32) for _ in range(NG)]
        + [pltpu.VMEM((KT,), jnp.int32), pltpu.VMEM((KT,), jnp.int32)]
        + [pltpu.VMEM((KT, D), jnp.float32)]
        + [pltpu.VMEM_SHARED((N_NODES, D), jnp.float32)]
        + [pltpu.SemaphoreType.DMA for _ in range(2 * NI + NG + 3)]
    ),
)


def _tc_root(x_ref, wrootT_ref, b_ref, out_ref):
    out_ref[...] = (jnp.dot(x_ref[...], wrootT_ref[...],
                            preferred_element_type=jnp.float32)
                    + b_ref[...])


_tc_root_call = pl.pallas_call(
    _tc_root,
    out_shape=jax.ShapeDtypeStruct((N_NODES, D), jnp.float32),
)


def _tc_finish(parts_ref, hroot_ref, wrT_ref, gamma_ref, beta_ref, out_ref):
    aggr = parts_ref[0] + parts_ref[1]
    h = (jnp.dot(aggr, wrT_ref[...], preferred_element_type=jnp.float32)
         + hroot_ref[...])
    mean = jnp.mean(h, axis=0, keepdims=True)
    d = h - mean
    var = jnp.mean(d * d, axis=0, keepdims=True)
    inv = lax.rsqrt(var + EPS)
    out_ref[...] = jnp.maximum(d * inv * gamma_ref[...] + beta_ref[...], 0.0)


_tc_finish_call = pl.pallas_call(
    _tc_finish,
    out_shape=jax.ShapeDtypeStruct((N_NODES, D), jnp.float32),
)


@jax.jit
def kernel(x, edge_index, batch, W_rel, b_rel, W_root, gamma, beta):
    edges = edge_index.astype(jnp.int32).reshape(2 * N_EDGES)
    hroot = _tc_root_call(x, W_root.T, b_rel.reshape(1, D))
    parts = _sc_aggregate_call(edges, x)
    return _tc_finish_call(parts, hroot, W_rel.T,
                           gamma.reshape(1, D), beta.reshape(1, D))


# speedup vs baseline: 1.1833x; 1.0051x over previous
import functools

import jax
import jax.numpy as jnp
from jax import lax
from jax.experimental import pallas as pl
from jax.experimental.pallas import tpu as pltpu
from jax.experimental.pallas import tpu_sc as plsc

N_NODES = 10000
N_EDGES = 320000
D = 128
EPS = 1e-5

NC = 2
NS = 16
NW = NC * NS
E_PER_TILE = N_EDGES // NW
K = 128
NCHUNK = E_PER_TILE // K
KT = E_PER_TILE - NCHUNK * K
NI = 4
NG = 2
ROWS_MAIN = 624
ROWS_TAIL_OFF = ROWS_MAIN * NS
ROWS_TAIL = N_NODES - ROWS_TAIL_OFF


def _sc_aggregate(edges_hbm, x_hbm, out_hbm,
                  iv0, iv1, iv2, iv3, dv0, dv1, dv2, dv3,
                  rows0, rows1, iv_t, dv_t, rows_t, aggr_sh,
                  is0, is1, is2, is3, ds0, ds1, ds2, ds3, gs0, gs1,
                  ist, dst_s, gst):
    ivs = (iv0, iv1, iv2, iv3)
    dvs = (dv0, dv1, dv2, dv3)
    isems = (is0, is1, is2, is3)
    dsems = (ds0, ds1, ds2, ds3)
    rows = (rows0, rows1)
    gsems = (gs0, gs1)

    c = lax.axis_index("c")
    s = lax.axis_index("s")
    wid = s * NC + c
    sbase = wid * E_PER_TILE
    dbase = N_EDGES + wid * E_PER_TILE

    def fire_idx(i, p):
        pltpu.async_copy(edges_hbm.at[pl.ds(sbase + K * i, K)],
                         ivs[p], isems[p])
        pltpu.async_copy(edges_hbm.at[pl.ds(dbase + K * i, K)],
                         dvs[p], dsems[p])

    def drain_isem(p):
        pltpu.make_async_copy(edges_hbm.at[pl.ds(0, K)], ivs[p],
                              isems[p]).wait()

    def drain_dsem(p):
        pltpu.make_async_copy(edges_hbm.at[pl.ds(0, K)], dvs[p],
                              dsems[p]).wait()

    def fire_gather(p):
        pltpu.async_copy(x_hbm.at[ivs[p % NI]], rows[p % NG], gsems[p % NG])

    def drain_gsem(p):
        pltpu.make_async_copy(x_hbm.at[pl.ds(0, K)], rows[p % NG],
                              gsems[p % NG]).wait()

    def scatter(p):
        pltpu.sync_copy(rows[p % NG], aggr_sh.at[dvs[p % NI]], add=True)

    def it(i, im, n=NCHUNK):
        if isinstance(i, int):
            if i + 1 < n:
                drain_isem((im + 1) % NI)
                fire_gather(im + 1)
        else:
            drain_isem((im + 1) % NI)
            fire_gather(im + 1)
        drain_gsem(im)
        drain_dsem(im % NI)
        scatter(im)
        if isinstance(i, int):
            if i + NI < n:
                fire_idx(i + NI, im % NI)
        else:
            fire_idx(i + NI, im % NI)

    for b in range(NI):
        fire_idx(b, b)
    pltpu.async_copy(edges_hbm.at[pl.ds(sbase + NCHUNK * K, KT)], iv_t, ist)
    pltpu.async_copy(edges_hbm.at[pl.ds(dbase + NCHUNK * K, KT)], dv_t, dst_s)
    drain_isem(0)
    fire_gather(0)
    pltpu.make_async_copy(edges_hbm.at[pl.ds(0, KT)], iv_t, ist).wait()
    pltpu.async_copy(x_hbm.at[iv_t], rows_t, gst)

    z16 = jnp.zeros((16,), jnp.float32)

    def zbody(r, carry):
        for j in range(D // 16):
            rows1[r, pl.ds(j * 16, 16)] = z16
        return carry

    lax.fori_loop(0, K, zbody, 0)

    r0 = s * ROWS_MAIN
    for t in range(4):
        pltpu.sync_copy(rows1, aggr_sh.at[pl.ds(r0 + K * t, K)])
    pltpu.sync_copy(rows1.at[pl.ds(0, 112)],
                    aggr_sh.at[pl.ds(r0 + 4 * K, 112)])

    @pl.when(s == NS - 1)
    def _():
        pltpu.sync_copy(rows1.at[pl.ds(0, ROWS_TAIL)],
                        aggr_sh.at[pl.ds(ROWS_TAIL_OFF, ROWS_TAIL)])

    plsc.subcore_barrier()

    def body(j, carry):
        i0 = j * NI
        for b in range(NI):
            it(i0 + b, b)
        return carry

    lax.fori_loop(0, 18, body, 0)

    for i in range(72, NCHUNK):
        it(i, i % NI)

    pltpu.make_async_copy(x_hbm.at[pl.ds(0, KT)], rows_t, gst).wait()
    pltpu.make_async_copy(edges_hbm.at[pl.ds(0, KT)], dv_t, dst_s).wait()
    pltpu.sync_copy(rows_t, aggr_sh.at[dv_t], add=True)

    plsc.subcore_barrier()

    pltpu.sync_copy(aggr_sh.at[pl.ds(r0, ROWS_MAIN)],
                    out_hbm.at[c, pl.ds(r0, ROWS_MAIN)])

    @pl.when(s == NS - 1)
    def _():
        pltpu.sync_copy(aggr_sh.at[pl.ds(ROWS_TAIL_OFF, ROWS_TAIL)],
                        out_hbm.at[c, pl.ds(ROWS_TAIL_OFF, ROWS_TAIL)])


_sc_aggregate_call = pl.kernel(
    _sc_aggregate,
    out_type=jax.ShapeDtypeStruct((NC, N_NODES, D), jnp.float32),
    mesh=plsc.VectorSubcoreMesh(core_axis_name="c", subcore_axis_name="s",
                                num_cores=NC, num_subcores=NS),
    scratch_types=(
        [pltpu.VMEM((K,), jnp.int32) for _ in range(NI)]
        + [pltpu.VMEM((K,), jnp.int32) for _ in range(NI)]
        + [pltpu.VMEM((K, D), jnp.float32) for _ in range(NG)]
        + [pltpu.VMEM((KT,), jnp.int32), pltpu.VMEM((KT,), jnp.int32)]
        + [pltpu.VMEM((KT, D), jnp.float32)]
        + [pltpu.VMEM_SHARED((N_NODES, D), jnp.float32)]
        + [pltpu.SemaphoreType.DMA for _ in range(2 * NI + NG + 3)]
    ),
)


def _tc_finish(parts_ref, x_ref, wrT_ref, b_ref, wrootT_ref,
               gamma_ref, beta_ref, out_ref):
    aggr = parts_ref[0] + parts_ref[1]
    h = (jnp.dot(aggr, wrT_ref[...], preferred_element_type=jnp.float32)
         + jnp.dot(x_ref[...], wrootT_ref[...],
                   preferred_element_type=jnp.float32)
         + b_ref[...])
    mean = jnp.mean(h, axis=0, keepdims=True)
    d = h - mean
    var = jnp.mean(d * d, axis=0, keepdims=True)
    inv = lax.rsqrt(var + EPS)
    out_ref[...] = jnp.maximum(d * inv * gamma_ref[...] + beta_ref[...], 0.0)


_tc_finish_call = pl.pallas_call(
    _tc_finish,
    out_shape=jax.ShapeDtypeStruct((N_NODES, D), jnp.float32),
)


@jax.jit
def kernel(x, edge_index, batch, W_rel, b_rel, W_root, gamma, beta):
    edges = edge_index.astype(jnp.int32).reshape(2 * N_EDGES)
    parts = _sc_aggregate_call(edges, x)
    return _tc_finish_call(parts, x, W_rel.T, b_rel.reshape(1, D),
                           W_root.T, gamma.reshape(1, D), beta.reshape(1, D))
